# Initial kernel scaffold; baseline (speedup 1.0000x reference)
import jax
import jax.numpy as jnp
from jax.experimental import pallas as pl


def kernel(x_ap, x_sta, edge_index_apap, edge_index_apsta, edge_index_staap, e_apap, e_apsta, e_staap, W_msg_apap, W_e_apap, W_self_apap, W_se_apap, W_de_apap, W_ee_apap, W_msg_apsta, W_e_apsta, W_self_apsta, W_se_apsta, W_de_apsta, W_ee_apsta, W_msg_staap, W_e_staap, W_self_staap, W_se_staap, W_de_staap, W_ee_staap):
    raise NotImplementedError("write your pallas kernel here")


# speedup vs baseline: 1.3862x; 1.3862x over previous
import functools

import jax
import jax.numpy as jnp
from jax import lax
from jax.experimental import pallas as pl
from jax.experimental.pallas import tpu as pltpu
from jax.experimental.pallas import tpu_sc as plsc

N = 10000
E = 160000
D = 128
DE = 16

NC = 2
NS = 16
NW = NC * NS

CHUNK = 128
NCH = 40
EPT = NCH * CHUNK
EPAD = NW * EPT

NPAD = 10112
ROWS_PT = NPAD // NS


def _tc1_body(xa_ref, xs_ref,
              wse1, wde1, wse2, wde2, wse3, wde3,
              s1, d1, s2, d2, s3, d3):
    xa = xa_ref[...]
    xs = xs_ref[...]
    dot = functools.partial(jnp.dot, preferred_element_type=jnp.float32)
    s1[...] = dot(xa, wse1[...])
    d1[...] = dot(xa, wde1[...])
    s2[...] = dot(xa, wse2[...])
    d2[...] = dot(xs, wde2[...])
    s3[...] = dot(xs, wse3[...])
    d3[...] = dot(xa, wde3[...])


def _tc2_body(a1, b1, a2, b2, a3, b3, xa_ref, xs_ref,
              wm1, we1, ws1, wm2, we2, ws2, wm3, we3, ws3,
              nap, nsta):
    xa = xa_ref[...]
    xs = xs_ref[...]
    dot = functools.partial(jnp.dot, preferred_element_type=jnp.float32)

    def dstd(a, b, wm, we, ws, x):
        acc = dot(a[0] + a[1], wm[...]) + dot(b[0] + b[1], we[...])
        return jnp.maximum(acc + dot(x, ws[...]), 0.0)

    d_apap = dstd(a1[...], b1[...], wm1, we1, ws1, xa)
    d_apsta = dstd(a2[...], b2[...], wm2, we2, ws2, xs)
    d_staap = dstd(a3[...], b3[...], wm3, we3, ws3, xa)
    nap[...] = 0.5 * (d_apap + d_staap)
    nsta[...] = d_apsta


def _tc3_body(sg1, dg1, e1, sg2, dg2, e2, sg3, dg3, e3,
              wee1, wee2, wee3, o1, o2, o3):
    dot = functools.partial(jnp.dot, preferred_element_type=jnp.float32)
    o1[...] = jnp.maximum(sg1[...] + dg1[...] + dot(e1[...], wee1[...]), 0.0)
    o2[...] = jnp.maximum(sg2[...] + dg2[...] + dot(e2[...], wee2[...]), 0.0)
    o3[...] = jnp.maximum(sg3[...] + dg3[...] + dot(e3[...], wee3[...]), 0.0)


def kernel(x_ap, x_sta, edge_index_apap, edge_index_apsta, edge_index_staap,
           e_apap, e_apsta, e_staap,
           W_msg_apap, W_e_apap, W_self_apap, W_se_apap, W_de_apap, W_ee_apap,
           W_msg_apsta, W_e_apsta, W_self_apsta, W_se_apsta, W_de_apsta, W_ee_apsta,
           W_msg_staap, W_e_staap, W_self_staap, W_se_staap, W_de_staap, W_ee_staap):
    f32 = jnp.float32

    nb = 10
    blk = N // nb
    row_spec = pl.BlockSpec((blk, D), lambda i: (i, 0))
    w_spec = pl.BlockSpec((D, DE), lambda i: (0, 0))
    out16 = pl.BlockSpec((blk, DE), lambda i: (i, 0))
    s1, d1, s2, d2, s3, d3 = pl.pallas_call(
        _tc1_body,
        grid=(nb,),
        in_specs=[row_spec, row_spec] + [w_spec] * 6,
        out_specs=[out16] * 6,
        out_shape=[jax.ShapeDtypeStruct((N, DE), f32)] * 6,
    )(x_ap, x_sta,
      W_se_apap, W_de_apap, W_se_apsta, W_de_apsta, W_se_staap, W_de_staap)

    pad_n = EPAD - E

    def prep_idx(ei):
        src = jnp.concatenate([ei[0], jnp.zeros((pad_n,), jnp.int32)])
        dst = jnp.concatenate([ei[1], jnp.full((pad_n,), N, jnp.int32)])
        return src.reshape(NW, NCH, CHUNK), dst.reshape(NW, NCH, CHUNK)

    def prep_e(e):
        return jnp.concatenate([e, jnp.zeros((pad_n, DE), f32)])

    def prep_tab(t):
        return jnp.concatenate([t, jnp.zeros((NPAD - N, DE), f32)])

    src1, dst1 = prep_idx(edge_index_apap)
    src2, dst2 = prep_idx(edge_index_apsta)
    src3, dst3 = prep_idx(edge_index_staap)
    zeros_a = jnp.zeros((ROWS_PT, D), f32)
    zeros_b = jnp.zeros((ROWS_PT, DE), f32)

    mesh = plsc.VectorSubcoreMesh(core_axis_name="c", subcore_axis_name="s")

    def make_sc():
        def body(x_src, s_tab, d_tab, e_pad, src_i, dst_i, za, zb,
                 a_out, b_out, sg_out, dg_out,
                 a_acc, b_acc, src_v, dst_v, xbuf, ebuf, sbuf, dbuf, sem):
            c = lax.axis_index("c")
            s = lax.axis_index("s")
            wid = c * NS + s
            pltpu.sync_copy(src_i.at[wid], src_v)
            pltpu.sync_copy(dst_i.at[wid], dst_v)
            row0 = s * ROWS_PT
            pltpu.sync_copy(za, a_acc.at[pl.ds(row0, ROWS_PT)])
            pltpu.sync_copy(zb, b_acc.at[pl.ds(row0, ROWS_PT)])
            plsc.subcore_barrier()

            def chunk(j, carry):
                si = src_v.at[j]
                di = dst_v.at[j]
                erow = wid * EPT + j * CHUNK
                pltpu.async_copy(x_src.at[si], xbuf, sem).wait()
                pltpu.sync_copy(xbuf, a_acc.at[di], add=True)
                pltpu.sync_copy(e_pad.at[pl.ds(erow, CHUNK)], ebuf)
                pltpu.sync_copy(ebuf, b_acc.at[di], add=True)
                pltpu.async_copy(s_tab.at[si], sbuf, sem).wait()
                pltpu.sync_copy(sbuf, sg_out.at[pl.ds(erow, CHUNK)])
                pltpu.async_copy(d_tab.at[di], dbuf, sem).wait()
                pltpu.sync_copy(dbuf, dg_out.at[pl.ds(erow, CHUNK)])
                return carry

            lax.fori_loop(0, NCH, chunk, 0)
            plsc.subcore_barrier()
            pltpu.sync_copy(a_acc.at[pl.ds(row0, ROWS_PT)],
                            a_out.at[c, pl.ds(row0, ROWS_PT)])
            pltpu.sync_copy(b_acc.at[pl.ds(row0, ROWS_PT)],
                            b_out.at[c, pl.ds(row0, ROWS_PT)])

        return pl.kernel(
            body,
            out_type=[
                jax.ShapeDtypeStruct((NC, NPAD, D), f32),
                jax.ShapeDtypeStruct((NC, NPAD, DE), f32),
                jax.ShapeDtypeStruct((EPAD, DE), f32),
                jax.ShapeDtypeStruct((EPAD, DE), f32),
            ],
            mesh=mesh,
            compiler_params=pltpu.CompilerParams(use_tc_tiling_on_sc=False),
            scratch_types=[
                pltpu.VMEM_SHARED((NPAD, D), f32),
                pltpu.VMEM_SHARED((NPAD, DE), f32),
                pltpu.VMEM((NCH, CHUNK), jnp.int32),
                pltpu.VMEM((NCH, CHUNK), jnp.int32),
                pltpu.VMEM((CHUNK, D), f32),
                pltpu.VMEM((CHUNK, DE), f32),
                pltpu.VMEM((CHUNK, DE), f32),
                pltpu.VMEM((CHUNK, DE), f32),
                pltpu.SemaphoreType.DMA,
            ],
        )

    sc = make_sc()
    a1o, b1o, sg1, dg1 = sc(x_ap, s1, prep_tab(d1), prep_e(e_apap),
                            src1, dst1, zeros_a, zeros_b)
    a2o, b2o, sg2, dg2 = sc(x_ap, s2, prep_tab(d2), prep_e(e_apsta),
                            src2, dst2, zeros_a, zeros_b)
    a3o, b3o, sg3, dg3 = sc(x_sta, s3, prep_tab(d3), prep_e(e_staap),
                            src3, dst3, zeros_a, zeros_b)

    a_spec = pl.BlockSpec((NC, blk, D), lambda i: (0, i, 0))
    b_spec = pl.BlockSpec((NC, blk, DE), lambda i: (0, i, 0))
    wdd = pl.BlockSpec((D, D), lambda i: (0, 0))
    wed = pl.BlockSpec((DE, D), lambda i: (0, 0))
    outD = pl.BlockSpec((blk, D), lambda i: (i, 0))
    nap, nsta = pl.pallas_call(
        _tc2_body,
        grid=(nb,),
        in_specs=[a_spec, b_spec, a_spec, b_spec, a_spec, b_spec,
                  row_spec, row_spec,
                  wdd, wed, wdd, wdd, wed, wdd, wdd, wed, wdd],
        out_specs=[outD, outD],
        out_shape=[jax.ShapeDtypeStruct((N, D), f32)] * 2,
    )(a1o, b1o, a2o, b2o, a3o, b3o, x_ap, x_sta,
      W_msg_apap, W_e_apap, W_self_apap,
      W_msg_apsta, W_e_apsta, W_self_apsta,
      W_msg_staap, W_e_staap, W_self_staap)

    eb = 2000
    neb = E // eb
    eg_spec = pl.BlockSpec((eb, DE), lambda i: (i, 0))
    wee_spec = pl.BlockSpec((DE, DE), lambda i: (0, 0))
    eo1, eo2, eo3 = pl.pallas_call(
        _tc3_body,
        grid=(neb,),
        in_specs=[eg_spec] * 9 + [wee_spec] * 3,
        out_specs=[eg_spec] * 3,
        out_shape=[jax.ShapeDtypeStruct((E, DE), f32)] * 3,
    )(sg1, dg1, e_apap, sg2, dg2, e_apsta, sg3, dg3, e_staap,
      W_ee_apap, W_ee_apsta, W_ee_staap)

    return nap, nsta, eo1, eo2, eo3


# speedup vs baseline: 1.5692x; 1.1321x over previous
import functools

import jax
import jax.numpy as jnp
from jax import lax
from jax.experimental import pallas as pl
from jax.experimental.pallas import tpu as pltpu
from jax.experimental.pallas import tpu_sc as plsc

N = 10000
E = 160000
D = 128
DE = 16

NC = 2
NS = 16
NW = NC * NS

CHUNK = 128
NCH = 40
EPT = NCH * CHUNK
EPAD = NW * EPT

NPAD = 10112
ROWS_PT = NPAD // NS


def _tc1_body(xa_ref, xs_ref,
              wse1, wde1, wse2, wde2, wse3, wde3,
              s1, d1, s2, d2, s3, d3):
    xa = xa_ref[...]
    xs = xs_ref[...]
    dot = functools.partial(jnp.dot, preferred_element_type=jnp.float32)
    s1[...] = dot(xa, wse1[...])
    d1[...] = dot(xa, wde1[...])
    s2[...] = dot(xa, wse2[...])
    d2[...] = dot(xs, wde2[...])
    s3[...] = dot(xs, wse3[...])
    d3[...] = dot(xa, wde3[...])


def _tc2_body(a1, b1, a2, b2, a3, b3, xa_ref, xs_ref,
              wm1, we1, ws1, wm2, we2, ws2, wm3, we3, ws3,
              nap, nsta):
    xa = xa_ref[...]
    xs = xs_ref[...]
    dot = functools.partial(jnp.dot, preferred_element_type=jnp.float32)

    def dstd(a, b, wm, we, ws, x):
        acc = dot(a[0] + a[1], wm[...]) + dot(b[0] + b[1], we[...])
        return jnp.maximum(acc + dot(x, ws[...]), 0.0)

    d_apap = dstd(a1[...], b1[...], wm1, we1, ws1, xa)
    d_apsta = dstd(a2[...], b2[...], wm2, we2, ws2, xs)
    d_staap = dstd(a3[...], b3[...], wm3, we3, ws3, xa)
    nap[...] = 0.5 * (d_apap + d_staap)
    nsta[...] = d_apsta


def _tc3_body(sg1, dg1, e1, sg2, dg2, e2, sg3, dg3, e3,
              wee1, wee2, wee3, o1, o2, o3):
    dot = functools.partial(jnp.dot, preferred_element_type=jnp.float32)
    o1[...] = jnp.maximum(sg1[...] + dg1[...] + dot(e1[...], wee1[...]), 0.0)
    o2[...] = jnp.maximum(sg2[...] + dg2[...] + dot(e2[...], wee2[...]), 0.0)
    o3[...] = jnp.maximum(sg3[...] + dg3[...] + dot(e3[...], wee3[...]), 0.0)


def kernel(x_ap, x_sta, edge_index_apap, edge_index_apsta, edge_index_staap,
           e_apap, e_apsta, e_staap,
           W_msg_apap, W_e_apap, W_self_apap, W_se_apap, W_de_apap, W_ee_apap,
           W_msg_apsta, W_e_apsta, W_self_apsta, W_se_apsta, W_de_apsta, W_ee_apsta,
           W_msg_staap, W_e_staap, W_self_staap, W_se_staap, W_de_staap, W_ee_staap):
    f32 = jnp.float32

    nb = 10
    blk = N // nb
    row_spec = pl.BlockSpec((blk, D), lambda i: (i, 0))
    w_spec = pl.BlockSpec((D, DE), lambda i: (0, 0))
    out16 = pl.BlockSpec((blk, DE), lambda i: (i, 0))
    s1, d1, s2, d2, s3, d3 = pl.pallas_call(
        _tc1_body,
        grid=(nb,),
        in_specs=[row_spec, row_spec] + [w_spec] * 6,
        out_specs=[out16] * 6,
        out_shape=[jax.ShapeDtypeStruct((N, DE), f32)] * 6,
    )(x_ap, x_sta,
      W_se_apap, W_de_apap, W_se_apsta, W_de_apsta, W_se_staap, W_de_staap)

    pad_n = EPAD - E

    def prep_idx(ei):
        src = jnp.concatenate([ei[0], jnp.zeros((pad_n,), jnp.int32)])
        dst = jnp.concatenate([ei[1], jnp.full((pad_n,), N, jnp.int32)])
        return src.reshape(NW, NCH, CHUNK), dst.reshape(NW, NCH, CHUNK)

    def prep_e(e):
        return jnp.concatenate([e, jnp.zeros((pad_n, DE), f32)])

    def prep_tab(t):
        return jnp.concatenate([t, jnp.zeros((NPAD - N, DE), f32)])

    src1, dst1 = prep_idx(edge_index_apap)
    src2, dst2 = prep_idx(edge_index_apsta)
    src3, dst3 = prep_idx(edge_index_staap)
    zeros_a = jnp.zeros((ROWS_PT, D), f32)
    zeros_b = jnp.zeros((ROWS_PT, DE), f32)

    mesh = plsc.VectorSubcoreMesh(core_axis_name="c", subcore_axis_name="s")

    def make_p1():
        def body(xa, xs, si1, di1, si2, di2, si3, di3, za,
                 a1_out, a2_out, a3_out,
                 a_acc, src_v, dst_v, xb, gx, tx):
            c = lax.axis_index("c")
            s = lax.axis_index("s")
            wid = c * NS + s
            row0 = s * ROWS_PT

            def do_rel(x_src, src_i, dst_i, a_out):
                pltpu.sync_copy(src_i.at[wid], src_v)
                pltpu.sync_copy(dst_i.at[wid], dst_v)
                pltpu.sync_copy(za, a_acc.at[pl.ds(row0, ROWS_PT)])
                plsc.subcore_barrier()

                def gather(j, b):
                    return pltpu.make_async_copy(
                        x_src.at[src_v.at[j]], xb.at[b], gx)

                def scat(j, b):
                    return pltpu.make_async_copy(
                        xb.at[b], a_acc.at[dst_v.at[j]], tx)

                gather(0, 0).start()

                @pl.loop(0, NCH, step=2)
                def _(j):
                    for b in (0, 1):
                        jj = j + b
                        gather(jj, b).wait()

                        @pl.when(jj > 0)
                        def _():
                            scat(jj - 1, 1 - b).wait()

                        scat(jj, b).start(add=True)

                        @pl.when(jj + 1 < NCH)
                        def _():
                            gather(jj + 1, 1 - b).start()

                scat(NCH - 1, (NCH - 1) & 1).wait()
                plsc.subcore_barrier()
                pltpu.sync_copy(a_acc.at[pl.ds(row0, ROWS_PT)],
                                a_out.at[c, pl.ds(row0, ROWS_PT)])

            do_rel(xa, si1, di1, a1_out)
            do_rel(xa, si2, di2, a2_out)
            do_rel(xs, si3, di3, a3_out)

        return pl.kernel(
            body,
            out_type=[jax.ShapeDtypeStruct((NC, NPAD, D), f32)] * 3,
            mesh=mesh,
            compiler_params=pltpu.CompilerParams(use_tc_tiling_on_sc=False),
            scratch_types=[
                pltpu.VMEM_SHARED((NPAD, D), f32),
                pltpu.VMEM((NCH, CHUNK), jnp.int32),
                pltpu.VMEM((NCH, CHUNK), jnp.int32),
                pltpu.VMEM((2, CHUNK, D), f32),
                pltpu.SemaphoreType.DMA,
                pltpu.SemaphoreType.DMA,
            ],
        )

    def make_p2():
        def body(st1, dt1, ep1, st2, dt2, ep2, st3, dt3, ep3,
                 si1, di1, si2, di2, si3, di3, zb,
                 b1_out, b2_out, b3_out, sg1_o, dg1_o, sg2_o, dg2_o, sg3_o, dg3_o,
                 b_acc, src_v, dst_v, eb, sb, db, ge, gs, gd, te, ts, td):
            c = lax.axis_index("c")
            s = lax.axis_index("s")
            wid = c * NS + s
            row0 = s * ROWS_PT

            def do_rel(s_tab, d_tab, e_pad, src_i, dst_i, b_out, sg_out, dg_out):
                pltpu.sync_copy(src_i.at[wid], src_v)
                pltpu.sync_copy(dst_i.at[wid], dst_v)
                pltpu.sync_copy(zb, b_acc.at[pl.ds(row0, ROWS_PT)])
                plsc.subcore_barrier()

                def gathers(j, b):
                    erow = wid * EPT + j * CHUNK
                    return (
                        pltpu.make_async_copy(
                            e_pad.at[pl.ds(erow, CHUNK)], eb.at[b], ge),
                        pltpu.make_async_copy(
                            s_tab.at[src_v.at[j]], sb.at[b], gs),
                        pltpu.make_async_copy(
                            d_tab.at[dst_v.at[j]], db.at[b], gd),
                    )

                def stores(j, b):
                    erow = wid * EPT + j * CHUNK
                    return (
                        (pltpu.make_async_copy(
                            eb.at[b], b_acc.at[dst_v.at[j]], te), True),
                        (pltpu.make_async_copy(
                            sb.at[b], sg_out.at[pl.ds(erow, CHUNK)], ts), False),
                        (pltpu.make_async_copy(
                            db.at[b], dg_out.at[pl.ds(erow, CHUNK)], td), False),
                    )

                for dsc in gathers(0, 0):
                    dsc.start()

                @pl.loop(0, NCH, step=2)
                def _(j):
                    for b in (0, 1):
                        jj = j + b
                        for dsc in gathers(jj, b):
                            dsc.wait()

                        @pl.when(jj > 0)
                        def _():
                            for dsc, _add in stores(jj - 1, 1 - b):
                                dsc.wait()

                        for dsc, _add in stores(jj, b):
                            dsc.start(add=_add)

                        @pl.when(jj + 1 < NCH)
                        def _():
                            for dsc in gathers(jj + 1, 1 - b):
                                dsc.start()

                for dsc, _add in stores(NCH - 1, (NCH - 1) & 1):
                    dsc.wait()
                plsc.subcore_barrier()
                pltpu.sync_copy(b_acc.at[pl.ds(row0, ROWS_PT)],
                                b_out.at[c, pl.ds(row0, ROWS_PT)])

            do_rel(st1, dt1, ep1, si1, di1, b1_out, sg1_o, dg1_o)
            do_rel(st2, dt2, ep2, si2, di2, b2_out, sg2_o, dg2_o)
            do_rel(st3, dt3, ep3, si3, di3, b3_out, sg3_o, dg3_o)

        return pl.kernel(
            body,
            out_type=[jax.ShapeDtypeStruct((NC, NPAD, DE), f32)] * 3
                     + [jax.ShapeDtypeStruct((EPAD, DE), f32)] * 6,
            mesh=mesh,
            compiler_params=pltpu.CompilerParams(use_tc_tiling_on_sc=False),
            scratch_types=[
                pltpu.VMEM_SHARED((NPAD, DE), f32),
                pltpu.VMEM((NCH, CHUNK), jnp.int32),
                pltpu.VMEM((NCH, CHUNK), jnp.int32),
                pltpu.VMEM((2, CHUNK, DE), f32),
                pltpu.VMEM((2, CHUNK, DE), f32),
                pltpu.VMEM((2, CHUNK, DE), f32),
            ] + [pltpu.SemaphoreType.DMA] * 6,
        )

    a1o, a2o, a3o = make_p1()(x_ap, x_sta, src1, dst1, src2, dst2, src3, dst3,
                              zeros_a)
    (b1o, b2o, b3o, sg1, dg1, sg2, dg2, sg3, dg3) = make_p2()(
        s1, prep_tab(d1), prep_e(e_apap),
        s2, prep_tab(d2), prep_e(e_apsta),
        s3, prep_tab(d3), prep_e(e_staap),
        src1, dst1, src2, dst2, src3, dst3, zeros_b)

    a_spec = pl.BlockSpec((NC, blk, D), lambda i: (0, i, 0))
    b_spec = pl.BlockSpec((NC, blk, DE), lambda i: (0, i, 0))
    wdd = pl.BlockSpec((D, D), lambda i: (0, 0))
    wed = pl.BlockSpec((DE, D), lambda i: (0, 0))
    outD = pl.BlockSpec((blk, D), lambda i: (i, 0))
    nap, nsta = pl.pallas_call(
        _tc2_body,
        grid=(nb,),
        in_specs=[a_spec, b_spec, a_spec, b_spec, a_spec, b_spec,
                  row_spec, row_spec,
                  wdd, wed, wdd, wdd, wed, wdd, wdd, wed, wdd],
        out_specs=[outD, outD],
        out_shape=[jax.ShapeDtypeStruct((N, D), f32)] * 2,
    )(a1o, b1o, a2o, b2o, a3o, b3o, x_ap, x_sta,
      W_msg_apap, W_e_apap, W_self_apap,
      W_msg_apsta, W_e_apsta, W_self_apsta,
      W_msg_staap, W_e_staap, W_self_staap)

    eb = 2000
    neb = E // eb
    eg_spec = pl.BlockSpec((eb, DE), lambda i: (i, 0))
    wee_spec = pl.BlockSpec((DE, DE), lambda i: (0, 0))
    eo1, eo2, eo3 = pl.pallas_call(
        _tc3_body,
        grid=(neb,),
        in_specs=[eg_spec] * 9 + [wee_spec] * 3,
        out_specs=[eg_spec] * 3,
        out_shape=[jax.ShapeDtypeStruct((E, DE), f32)] * 3,
    )(sg1, dg1, e_apap, sg2, dg2, e_apsta, sg3, dg3, e_staap,
      W_ee_apap, W_ee_apsta, W_ee_staap)

    return nap, nsta, eo1, eo2, eo3
